# baseline (speedup 1.0000x reference)
import functools

import jax
import jax.numpy as jnp
from jax import lax
from jax.experimental import pallas as pl
from jax.experimental.pallas import tpu as pltpu
from jax.experimental.pallas import tpu_sc as plsc

NC, NS = 2, 16
NW = NC * NS
CHG = 128
NBUF = 4
CHS = 128
NBUFS = 2
K = 4
BE = 2000
BN = 1000


def _asinh(x):
    return jnp.sign(x) * jnp.log(jnp.abs(x) + jnp.sqrt(x * x + 1.0))


def _node_embed_body(na_ref, nodes_ref, nW_ref, nb_ref, wsr_ref, ne_ref,
                     proj_ref):
    x = nodes_ref[...]
    col = lax.broadcasted_iota(jnp.int32, x.shape, 1)
    x = jnp.where(col < 4, 0.0, x)
    x = na_ref[0] * _asinh(na_ref[1] * x + na_ref[2]) + na_ref[3]
    ne = jnp.dot(x, nW_ref[...], preferred_element_type=jnp.float32) + nb_ref[...]
    ne_ref[...] = ne
    proj_ref[...] = jnp.dot(ne, wsr_ref[...], preferred_element_type=jnp.float32)


def _node_embed(na, nodes, nW, nb2, wsr):
    n, _ = nodes.shape
    lat = nW.shape[1]
    return pl.pallas_call(
        _node_embed_body,
        out_shape=[jax.ShapeDtypeStruct((n, lat), jnp.float32),
                   jax.ShapeDtypeStruct((n, lat), jnp.float32)],
        in_specs=[pl.BlockSpec(memory_space=pltpu.SMEM)]
        + [pl.BlockSpec(memory_space=pltpu.VMEM)] * 4,
    )(na, nodes, nW, nb2, wsr)


def _sc_gather(table, idx3, nch, lat):
    epw = nch * CHG
    e_pad = NW * epw
    mesh = plsc.VectorSubcoreMesh(core_axis_name="c", subcore_axis_name="s")

    @functools.partial(
        pl.kernel,
        out_type=jax.ShapeDtypeStruct((e_pad, lat), jnp.float32),
        mesh=mesh,
        scratch_types=[pltpu.VMEM((nch, CHG), jnp.int32)]
        + [pltpu.VMEM((CHG, lat), jnp.float32)] * NBUF
        + [pltpu.SemaphoreType.DMA] * (2 * NBUF),
    )
    def gather_k(table_hbm, idx_hbm, out_hbm, idx_v, *bufs_sems):
        rows = bufs_sems[:NBUF]
        gsem = bufs_sems[NBUF:2 * NBUF]
        osem = bufs_sems[2 * NBUF:]
        c = lax.axis_index("c")
        s = lax.axis_index("s")
        wid = s * NC + c
        base = wid * epw
        pltpu.sync_copy(idx_hbm.at[wid], idx_v)
        ngrp = nch // NBUF

        def group(t, carry):
            j0 = t * NBUF

            @pl.when(t > 0)
            def _():
                for b in range(NBUF):
                    pltpu.make_async_copy(
                        rows[b], out_hbm.at[pl.ds(base, CHG)], osem[b]).wait()

            for b in range(NBUF):
                pltpu.async_copy(table_hbm.at[idx_v.at[j0 + b]], rows[b],
                                 gsem[b])
            for b in range(NBUF):
                pltpu.make_async_copy(out_hbm.at[pl.ds(base, CHG)], rows[b],
                                      gsem[b]).wait()
                pltpu.async_copy(rows[b],
                                 out_hbm.at[pl.ds(base + (j0 + b) * CHG, CHG)],
                                 osem[b])
            return carry

        lax.fori_loop(0, ngrp, group, 0)
        for b in range(NBUF):
            pltpu.make_async_copy(rows[b], out_hbm.at[pl.ds(base, CHG)],
                                  osem[b]).wait()

    return gather_k(table, idx3)


def _edge_mlp_body(ea_ref, edges_ref, gath_ref, wee_ref, ew1_ref, ce_ref,
                   eb1_ref, eu_ref, stats_ref):
    i = pl.program_id(0)
    x = ea_ref[0] * _asinh(ea_ref[1] * edges_ref[...] + ea_ref[2]) + ea_ref[3]
    pre = (jnp.dot(x, wee_ref[...], preferred_element_type=jnp.float32)
           + gath_ref[...] + ce_ref[...])
    h = jax.nn.gelu(pre)
    eu = jnp.dot(h, ew1_ref[...], preferred_element_type=jnp.float32) + eb1_ref[...]
    eu_ref[...] = eu
    s = jnp.sum(eu, axis=0, keepdims=True)
    sq = jnp.sum(eu * eu, axis=0, keepdims=True)
    mx = jnp.max(eu, axis=0, keepdims=True)

    @pl.when(i == 0)
    def _():
        stats_ref[0:1, :] = s
        stats_ref[1:2, :] = sq
        stats_ref[2:3, :] = mx

    @pl.when(i > 0)
    def _():
        stats_ref[0:1, :] += s
        stats_ref[1:2, :] += sq
        stats_ref[2:3, :] = jnp.maximum(stats_ref[2:3, :], mx)


def _edge_mlp(ea, edges, gathered, wee, ew1, ce, eb1_2, koff, ec):
    _, de = edges.shape
    lat = wee.shape[1]
    grid = ec // BE
    return pl.pallas_call(
        _edge_mlp_body,
        grid=(grid,),
        out_shape=[jax.ShapeDtypeStruct((ec, lat), jnp.float32),
                   jax.ShapeDtypeStruct((8, lat), jnp.float32)],
        in_specs=[pl.BlockSpec(memory_space=pltpu.SMEM),
                  pl.BlockSpec((BE, de), lambda i: (i + koff, 0)),
                  pl.BlockSpec((BE, lat), lambda i: (i, 0)),
                  pl.BlockSpec((de, lat), lambda i: (0, 0)),
                  pl.BlockSpec((lat, lat), lambda i: (0, 0)),
                  pl.BlockSpec((1, lat), lambda i: (0, 0)),
                  pl.BlockSpec((1, lat), lambda i: (0, 0))],
        out_specs=[pl.BlockSpec((BE, lat), lambda i: (i, 0)),
                   pl.BlockSpec((8, lat), lambda i: (0, 0))],
    )(ea, edges, gathered, wee, ew1, ce, eb1_2)


def _sc_scatter(eu, idx3, zeros, n, ec, lat, nch):
    epw = nch * CHS
    z0 = (n // NS) // 8 * 8
    z1 = n - z0 * (NS - 1)
    mesh = plsc.VectorSubcoreMesh(core_axis_name="c", subcore_axis_name="s")

    @functools.partial(
        pl.kernel,
        out_type=jax.ShapeDtypeStruct((NC * n, lat), jnp.float32),
        mesh=mesh,
        scratch_types=[pltpu.VMEM((nch, CHS), jnp.int32),
                       pltpu.VMEM_SHARED((n + 8, lat), jnp.float32)]
        + [pltpu.VMEM((CHS, lat), jnp.float32)] * NBUFS
        + [pltpu.SemaphoreType.DMA] * (2 * NBUFS),
    )
    def scatter_k(eu_hbm, idx_hbm, zero_hbm, out_hbm, idx_v, acc_sh,
                  *bufs_sems):
        rows = bufs_sems[:NBUFS]
        rsem = bufs_sems[NBUFS:2 * NBUFS]
        ssem = bufs_sems[2 * NBUFS:]
        c = lax.axis_index("c")
        s = lax.axis_index("s")
        wid = s * NC + c
        base = wid * epw
        ngrp = nch // NBUFS

        @pl.when(s < NS - 1)
        def _():
            pltpu.sync_copy(zero_hbm.at[pl.ds(s * z0, z0)],
                            acc_sh.at[pl.ds(s * z0, z0)])

        @pl.when(s == NS - 1)
        def _():
            pltpu.sync_copy(zero_hbm.at[pl.ds((NS - 1) * z0, z1)],
                            acc_sh.at[pl.ds((NS - 1) * z0, z1)])

        plsc.subcore_barrier()
        pltpu.sync_copy(idx_hbm.at[wid], idx_v)

        def group(t, carry):
            j0 = t * NBUFS

            @pl.when(t > 0)
            def _():
                for b in range(NBUFS):
                    pltpu.make_async_copy(rows[b], acc_sh.at[pl.ds(0, CHS)],
                                          ssem[b]).wait()

            for b in range(NBUFS):
                off = jnp.minimum(base + (j0 + b) * CHS, ec - CHS)
                pltpu.async_copy(eu_hbm.at[pl.ds(off, CHS)], rows[b], rsem[b])
            for b in range(NBUFS):
                pltpu.make_async_copy(eu_hbm.at[pl.ds(0, CHS)], rows[b],
                                      rsem[b]).wait()
                pltpu.async_copy(rows[b], acc_sh.at[idx_v.at[j0 + b]],
                                 ssem[b], add=True)
            return carry

        lax.fori_loop(0, ngrp, group, 0)
        for b in range(NBUFS):
            pltpu.make_async_copy(rows[b], acc_sh.at[pl.ds(0, CHS)],
                                  ssem[b]).wait()
        plsc.subcore_barrier()

        @pl.when(s < NS - 1)
        def _():
            pltpu.sync_copy(acc_sh.at[pl.ds(s * z0, z0)],
                            out_hbm.at[pl.ds(c * n + s * z0, z0)])

        @pl.when(s == NS - 1)
        def _():
            pltpu.sync_copy(acc_sh.at[pl.ds((NS - 1) * z0, z1)],
                            out_hbm.at[pl.ds(c * n + (NS - 1) * z0, z1)])

    return scatter_k(eu, idx3, zeros)


def _node_update_body(*refs):
    den_ref = refs[0]
    ne_ref = refs[1]
    parts = refs[2:2 + K]
    (vwa_ref, vwsr_ref, cn_ref, vw1_ref, vb1_ref, stats_ref, ge_ref, uw0_ref,
     ub0_ref, uw1_ref, ub1_ref, uw2_ref, ub2_ref, out_ref, acc_ref) = refs[2 + K:]
    i = pl.program_id(0)
    ng = pl.num_programs(0)
    sagg = parts[0][0] + parts[0][1]
    for pk in parts[1:]:
        sagg = sagg + pk[0] + pk[1]
    pre = (jnp.dot(ne_ref[...], vwa_ref[...], preferred_element_type=jnp.float32)
           + jnp.dot(sagg, vwsr_ref[...], preferred_element_type=jnp.float32)
           + cn_ref[...])
    h = jax.nn.gelu(pre)
    nu = jnp.dot(h, vw1_ref[...], preferred_element_type=jnp.float32) + vb1_ref[...]
    ns = jnp.sum(nu, axis=0, keepdims=True)
    nsq = jnp.sum(nu * nu, axis=0, keepdims=True)
    nmx = jnp.max(nu, axis=0, keepdims=True)

    @pl.when(i == 0)
    def _():
        acc_ref[0:1, :] = ns
        acc_ref[1:2, :] = nsq
        acc_ref[2:3, :] = nmx

    @pl.when(i > 0)
    def _():
        acc_ref[0:1, :] += ns
        acc_ref[1:2, :] += nsq
        acc_ref[2:3, :] = jnp.maximum(acc_ref[2:3, :], nmx)

    @pl.when(i == ng - 1)
    def _():
        nn = den_ref[0]
        nedge = den_ref[1]
        nsum = acc_ref[0:1, :]
        nmean = nsum / nn
        nvar = acc_ref[1:2, :] / nn - nmean * nmean
        nattr = _asinh(jnp.concatenate(
            [nsum, nmean, nvar, acc_ref[2:3, :]], axis=1))
        esum = stats_ref[0:1, :]
        esq = stats_ref[1:2, :]
        emx = stats_ref[2:3, :]
        for k in range(1, K):
            esum = esum + stats_ref[8 * k:8 * k + 1, :]
            esq = esq + stats_ref[8 * k + 1:8 * k + 2, :]
            emx = jnp.maximum(emx, stats_ref[8 * k + 2:8 * k + 3, :])
        emean = esum / nedge
        evar = esq / nedge - emean * emean
        eattr = _asinh(jnp.concatenate([esum, emean, evar, emx], axis=1))
        u = jnp.concatenate([nattr, eattr, ge_ref[...]], axis=1)
        x1 = jax.nn.gelu(jnp.dot(u, uw0_ref[...],
                                 preferred_element_type=jnp.float32)
                         + ub0_ref[...])
        x2 = jax.nn.gelu(jnp.dot(x1, uw1_ref[...],
                                 preferred_element_type=jnp.float32)
                         + ub1_ref[...])
        out_ref[...] = (jnp.dot(x2, uw2_ref[...],
                                preferred_element_type=jnp.float32)
                        + ub2_ref[...])


def _node_update(den, nodes_e, parts_list, vwa, vwsr, cn, vw1, vb1_2, stats,
                 ge, uw0, ub0_2, uw1, ub1_2, uw2, ub2_2):
    n, lat = nodes_e.shape
    n_out = uw2.shape[1]
    grid = n // BN
    const = lambda shape: pl.BlockSpec(shape, lambda i: tuple(0 for _ in shape))
    return pl.pallas_call(
        _node_update_body,
        grid=(grid,),
        out_shape=jax.ShapeDtypeStruct((1, n_out), jnp.float32),
        in_specs=[pl.BlockSpec(memory_space=pltpu.SMEM),
                  pl.BlockSpec((BN, lat), lambda i: (i, 0))]
        + [pl.BlockSpec((NC, BN, lat), lambda i: (0, i, 0))] * K
        + [const((lat, lat)), const((lat, lat)), const((1, lat)),
           const((lat, lat)), const((1, lat)), const((8 * K, lat)),
           const((1, lat)), const(tuple(uw0.shape)), const((1, lat)),
           const((lat, lat)), const((1, lat)), const(tuple(uw2.shape)),
           const((1, n_out))],
        out_specs=pl.BlockSpec((1, n_out), lambda i: (0, 0)),
        scratch_shapes=[pltpu.VMEM((8, lat), jnp.float32)],
    )(den, nodes_e, *parts_list, vwa, vwsr, cn, vw1, vb1_2, stats, ge,
      uw0, ub0_2, uw1, ub1_2, uw2, ub2_2)


def kernel(nodes, edges, senders, receivers, n_node, n_edge, na, nW, nb, ea,
           eW, eb, ga, gW, gb, eW0, eb0, eW1, eb1, vW0, vb0, vW1, vb1, uW0,
           ub0, uW1, ub1, uW2, ub2):
    f32 = jnp.float32
    n, _ = nodes.shape
    e, _ = edges.shape
    lat = nW.shape[1]

    g = jnp.concatenate([jnp.arcsinh(n_node.astype(f32)),
                         jnp.arcsinh(n_edge.astype(f32))]).reshape(1, 2)
    glob_e = (ga[0] * jnp.arcsinh(ga[1] * g + ga[2]) + ga[3]) @ gW + gb
    eW0a = eW0[0:lat]
    wsr_e = eW0[lat:2 * lat] + eW0[2 * lat:3 * lat]
    eW0g = eW0[3 * lat:4 * lat]
    wee = eW @ eW0a
    ce = (eb @ eW0a + glob_e[0] @ eW0g + eb0).reshape(1, lat)
    vwa = vW0[0:lat]
    vwsr = vW0[lat:2 * lat] + vW0[2 * lat:3 * lat]
    vW0g = vW0[3 * lat:4 * lat]
    cn = (glob_e[0] @ vW0g + vb0).reshape(1, lat)
    den = jnp.maximum(jnp.concatenate([n_node, n_edge]).astype(f32), 1.0)

    nodes_e, proj = _node_embed(na, nodes, nW, nb.reshape(1, lat), wsr_e)

    zeros = jnp.zeros((n, lat), f32)
    ec = e // K
    nch = (-(-ec // (NW * CHS)) + NBUFS - 1) // NBUFS * NBUFS
    nch = (nch + NBUF - 1) // NBUF * NBUF
    npad = NW * nch * CHS - ec
    parts_list, stats_list = [], []
    for k in range(K):
        sndk = senders[k * ec:(k + 1) * ec]
        gidx = jnp.concatenate([sndk, jnp.zeros((npad,), jnp.int32)]
                               ).reshape(NW, nch, CHG)
        gath_k = _sc_gather(proj, gidx, nch, lat)
        eu_k, st_k = _edge_mlp(ea, edges, gath_k, wee, eW1, ce,
                               eb1.reshape(1, lat), k * (ec // BE), ec)
        sidx = jnp.concatenate([sndk, jnp.full((npad,), n, jnp.int32)]
                               ).reshape(NW, nch, CHS)
        parts_list.append(
            _sc_scatter(eu_k, sidx, zeros, n, ec, lat, nch).reshape(NC, n, lat))
        stats_list.append(st_k)
    stats = jnp.concatenate(stats_list, axis=0)

    out = _node_update(den, nodes_e, parts_list, vwa, vwsr, cn,
                       vW1, vb1.reshape(1, lat), stats, glob_e,
                       uW0, ub0.reshape(1, -1), uW1, ub1.reshape(1, -1),
                       uW2, ub2.reshape(1, -1))
    return out.reshape(-1)


# speedup vs baseline: 1.7263x; 1.7263x over previous
import functools

import jax
import jax.numpy as jnp
from jax import lax
from jax.experimental import pallas as pl
from jax.experimental.pallas import tpu as pltpu
from jax.experimental.pallas import tpu_sc as plsc

NC, NS = 2, 16
NW = NC * NS
CH = 80


def _asinh(x):
    return jnp.sign(x) * jnp.log(jnp.abs(x) + jnp.sqrt(x * x + 1.0))


def _node_embed_body(na_ref, nodes_ref, nW_ref, nb_ref, wsr_ref, ne_ref, proj_ref):
    x = nodes_ref[...]
    col = lax.broadcasted_iota(jnp.int32, x.shape, 1)
    x = jnp.where(col < 4, 0.0, x)
    x = na_ref[0] * _asinh(na_ref[1] * x + na_ref[2]) + na_ref[3]
    ne = jnp.dot(x, nW_ref[...], preferred_element_type=jnp.float32) + nb_ref[...]
    ne_ref[...] = ne
    proj_ref[...] = jnp.dot(ne, wsr_ref[...], preferred_element_type=jnp.float32)


def _node_embed(na, nodes, nW, nb2, wsr):
    n, d = nodes.shape
    lat = nW.shape[1]
    return pl.pallas_call(
        _node_embed_body,
        out_shape=[jax.ShapeDtypeStruct((n, lat), jnp.float32),
                   jax.ShapeDtypeStruct((n, lat), jnp.float32)],
        in_specs=[pl.BlockSpec(memory_space=pltpu.SMEM),
                  pl.BlockSpec(memory_space=pltpu.VMEM),
                  pl.BlockSpec(memory_space=pltpu.VMEM),
                  pl.BlockSpec(memory_space=pltpu.VMEM),
                  pl.BlockSpec(memory_space=pltpu.VMEM)],
    )(na, nodes, nW, nb2, wsr)


NBUF = 5


def _sc_gather(proj, snd2, e, lat):
    epw = e // NW
    nch = epw // CH
    ngrp = nch // NBUF
    mesh = plsc.VectorSubcoreMesh(core_axis_name="c", subcore_axis_name="s")

    @functools.partial(
        pl.kernel,
        out_type=jax.ShapeDtypeStruct((e, lat), jnp.float32),
        mesh=mesh,
        scratch_types=[pltpu.VMEM((nch, CH), jnp.int32)]
        + [pltpu.VMEM((CH, lat), jnp.float32)] * NBUF
        + [pltpu.SemaphoreType.DMA] * (2 * NBUF),
    )
    def gather_k(table_hbm, idx_hbm, out_hbm, idx_v, *bufs_sems):
        rows = bufs_sems[:NBUF]
        gsem = bufs_sems[NBUF:2 * NBUF]
        osem = bufs_sems[2 * NBUF:]
        c = lax.axis_index("c")
        s = lax.axis_index("s")
        wid = s * NC + c
        base = wid * epw
        pltpu.sync_copy(idx_hbm.at[wid], idx_v)

        def group(t, carry):
            j0 = t * NBUF

            for b in range(NBUF):
                @pl.when(t > 0)
                def _(b=b):
                    pltpu.make_async_copy(
                        rows[b], out_hbm.at[pl.ds(base, CH)], osem[b]).wait()

                pltpu.async_copy(table_hbm.at[idx_v.at[j0 + b]], rows[b],
                                 gsem[b])
            for b in range(NBUF):
                pltpu.make_async_copy(out_hbm.at[pl.ds(base, CH)], rows[b],
                                      gsem[b]).wait()
                pltpu.async_copy(rows[b],
                                 out_hbm.at[pl.ds(base + (j0 + b) * CH, CH)],
                                 osem[b])
            return carry

        lax.fori_loop(0, ngrp, group, 0)
        for b in range(NBUF):
            pltpu.make_async_copy(rows[b], out_hbm.at[pl.ds(base, CH)],
                                  osem[b]).wait()

    return gather_k(proj, snd2)


def _edge_mlp_body(ea_ref, edges_ref, gath_ref, wee_ref, ew1_ref, ce_ref,
                   eb1_ref, eu_ref, stats_ref):
    i = pl.program_id(0)
    x = ea_ref[0] * _asinh(ea_ref[1] * edges_ref[...] + ea_ref[2]) + ea_ref[3]
    pe = jnp.dot(x, wee_ref[...], preferred_element_type=jnp.float32)
    pre = (pe.reshape(gath_ref.shape) + gath_ref[...] + ce_ref[...])
    h = jax.nn.gelu(pre)
    eu = jnp.dot(h, ew1_ref[...], preferred_element_type=jnp.float32) + eb1_ref[...]
    eu_ref[...] = eu
    s = jnp.sum(eu, axis=0, keepdims=True)
    sq = jnp.sum(eu * eu, axis=0, keepdims=True)
    mx = jnp.max(eu, axis=0, keepdims=True)

    @pl.when(i == 0)
    def _():
        stats_ref[0:1, :] = s
        stats_ref[1:2, :] = sq
        stats_ref[2:3, :] = mx

    @pl.when(i > 0)
    def _():
        stats_ref[0:1, :] += s
        stats_ref[1:2, :] += sq
        stats_ref[2:3, :] = jnp.maximum(stats_ref[2:3, :], mx)


def _edge_mlp(ea, edges_pk, gathered, wee8, ew1, ce, eb1_2, be, koff, ec):
    lat = ew1.shape[0]
    grid = ec // be
    bp = be // 8
    return pl.pallas_call(
        _edge_mlp_body,
        grid=(grid,),
        out_shape=[jax.ShapeDtypeStruct((ec, lat), jnp.float32),
                   jax.ShapeDtypeStruct((8, lat), jnp.float32)],
        in_specs=[pl.BlockSpec(memory_space=pltpu.SMEM),
                  pl.BlockSpec((bp, 128), lambda i: (i + koff, 0)),
                  pl.BlockSpec((be, lat), lambda i: (i, 0)),
                  pl.BlockSpec((128, 8 * lat), lambda i: (0, 0)),
                  pl.BlockSpec((lat, lat), lambda i: (0, 0)),
                  pl.BlockSpec((1, lat), lambda i: (0, 0)),
                  pl.BlockSpec((1, lat), lambda i: (0, 0))],
        out_specs=[pl.BlockSpec((be, lat), lambda i: (i, 0)),
                   pl.BlockSpec((8, lat), lambda i: (0, 0))],
    )(ea, edges_pk, gathered, wee8, ew1, ce, eb1_2)


CHS = 128
NBUFS = 2


def _sc_scatter(eu, snd3, zeros, n, e, lat, nch):
    epw_pad = nch * CHS
    z0 = (n // NS) // 8 * 8
    z1 = n - z0 * (NS - 1)
    mesh = plsc.VectorSubcoreMesh(core_axis_name="c", subcore_axis_name="s")

    @functools.partial(
        pl.kernel,
        out_type=jax.ShapeDtypeStruct((NC * n, lat), jnp.float32),
        mesh=mesh,
        scratch_types=[pltpu.VMEM((nch, CHS), jnp.int32),
                       pltpu.VMEM_SHARED((n + 8, lat), jnp.float32)]
        + [pltpu.VMEM((CHS, lat), jnp.float32)] * NBUFS
        + [pltpu.SemaphoreType.DMA] * (2 * NBUFS),
    )
    def scatter_k(eu_hbm, idx_hbm, zero_hbm, out_hbm, idx_v, acc_sh,
                  *bufs_sems):
        rows = bufs_sems[:NBUFS]
        rsem = bufs_sems[NBUFS:2 * NBUFS]
        ssem = bufs_sems[2 * NBUFS:]
        c = lax.axis_index("c")
        s = lax.axis_index("s")
        wid = s * NC + c
        base = wid * epw_pad
        ngrp = nch // NBUFS

        @pl.when(s < NS - 1)
        def _():
            pltpu.sync_copy(zero_hbm.at[pl.ds(s * z0, z0)],
                            acc_sh.at[pl.ds(s * z0, z0)])

        @pl.when(s == NS - 1)
        def _():
            pltpu.sync_copy(zero_hbm.at[pl.ds((NS - 1) * z0, z1)],
                            acc_sh.at[pl.ds((NS - 1) * z0, z1)])

        plsc.subcore_barrier()
        pltpu.sync_copy(idx_hbm.at[wid], idx_v)

        def group(t, carry):
            j0 = t * NBUFS

            for b in range(NBUFS):
                @pl.when(t > 0)
                def _(b=b):
                    pltpu.make_async_copy(rows[b], acc_sh.at[pl.ds(0, CHS)],
                                          ssem[b]).wait()

                off = jnp.minimum(base + (j0 + b) * CHS, e - CHS)
                pltpu.async_copy(eu_hbm.at[pl.ds(off, CHS)], rows[b], rsem[b])
            for b in range(NBUFS):
                pltpu.make_async_copy(eu_hbm.at[pl.ds(0, CHS)], rows[b],
                                      rsem[b]).wait()
                pltpu.async_copy(rows[b], acc_sh.at[idx_v.at[j0 + b]],
                                 ssem[b], add=True)
            return carry

        lax.fori_loop(0, ngrp, group, 0)
        for b in range(NBUFS):
            pltpu.make_async_copy(rows[b], acc_sh.at[pl.ds(0, CHS)],
                                  ssem[b]).wait()
        plsc.subcore_barrier()

        @pl.when(s < NS - 1)
        def _():
            pltpu.sync_copy(acc_sh.at[pl.ds(s * z0, z0)],
                            out_hbm.at[pl.ds(c * n + s * z0, z0)])

        @pl.when(s == NS - 1)
        def _():
            pltpu.sync_copy(acc_sh.at[pl.ds((NS - 1) * z0, z1)],
                            out_hbm.at[pl.ds(c * n + (NS - 1) * z0, z1)])

    return scatter_k(eu, snd3, zeros)


K = 5
BN = 1000


def _node_update_body(*refs):
    den_ref = refs[0]
    ne_ref = refs[1]
    parts = refs[2:2 + K]
    (vwa_ref, vwsr_ref, cn_ref, vw1_ref, vb1_ref, stats_ref, ge_ref, uw0_ref,
     ub0_ref, uw1_ref, ub1_ref, uw2_ref, ub2_ref, out_ref, acc_ref) = refs[2 + K:]
    i = pl.program_id(0)
    ng = pl.num_programs(0)
    sagg = parts[0][0] + parts[0][1]
    for pk in parts[1:]:
        sagg = sagg + pk[0] + pk[1]
    pre = (jnp.dot(ne_ref[...], vwa_ref[...], preferred_element_type=jnp.float32)
           + jnp.dot(sagg, vwsr_ref[...], preferred_element_type=jnp.float32)
           + cn_ref[...])
    h = jax.nn.gelu(pre)
    nu = jnp.dot(h, vw1_ref[...], preferred_element_type=jnp.float32) + vb1_ref[...]
    ns = jnp.sum(nu, axis=0, keepdims=True)
    nsq = jnp.sum(nu * nu, axis=0, keepdims=True)
    nmx = jnp.max(nu, axis=0, keepdims=True)

    @pl.when(i == 0)
    def _():
        acc_ref[0:1, :] = ns
        acc_ref[1:2, :] = nsq
        acc_ref[2:3, :] = nmx

    @pl.when(i > 0)
    def _():
        acc_ref[0:1, :] += ns
        acc_ref[1:2, :] += nsq
        acc_ref[2:3, :] = jnp.maximum(acc_ref[2:3, :], nmx)

    @pl.when(i == ng - 1)
    def _():
        nn = den_ref[0]
        nedge = den_ref[1]
        nsum = acc_ref[0:1, :]
        nmean = nsum / nn
        nvar = acc_ref[1:2, :] / nn - nmean * nmean
        nattr = _asinh(jnp.concatenate(
            [nsum, nmean, nvar, acc_ref[2:3, :]], axis=1))
        esum = stats_ref[0:1, :]
        esq = stats_ref[1:2, :]
        emx = stats_ref[2:3, :]
        for k in range(1, K):
            esum = esum + stats_ref[8 * k:8 * k + 1, :]
            esq = esq + stats_ref[8 * k + 1:8 * k + 2, :]
            emx = jnp.maximum(emx, stats_ref[8 * k + 2:8 * k + 3, :])
        emean = esum / nedge
        evar = esq / nedge - emean * emean
        eattr = _asinh(jnp.concatenate([esum, emean, evar, emx], axis=1))
        u = jnp.concatenate([nattr, eattr, ge_ref[...]], axis=1)
        x1 = jax.nn.gelu(jnp.dot(u, uw0_ref[...],
                                 preferred_element_type=jnp.float32)
                         + ub0_ref[...])
        x2 = jax.nn.gelu(jnp.dot(x1, uw1_ref[...],
                                 preferred_element_type=jnp.float32)
                         + ub1_ref[...])
        out_ref[...] = (jnp.dot(x2, uw2_ref[...],
                                preferred_element_type=jnp.float32)
                        + ub2_ref[...])


def _node_update(den, nodes_e, parts_list, vwa, vwsr, cn, vw1, vb1_2, stats,
                 ge, uw0, ub0_2, uw1, ub1_2, uw2, ub2_2):
    n, lat = nodes_e.shape
    n_out = uw2.shape[1]
    grid = n // BN
    const = lambda shape: pl.BlockSpec(shape, lambda i: tuple(0 for _ in shape))
    return pl.pallas_call(
        _node_update_body,
        grid=(grid,),
        out_shape=jax.ShapeDtypeStruct((1, n_out), jnp.float32),
        in_specs=[pl.BlockSpec(memory_space=pltpu.SMEM),
                  pl.BlockSpec((BN, lat), lambda i: (i, 0))]
        + [pl.BlockSpec((NC, BN, lat), lambda i: (0, i, 0))] * K
        + [const((lat, lat)), const((lat, lat)), const((1, lat)),
           const((lat, lat)), const((1, lat)), const((8 * K, lat)),
           const((1, lat)), const(tuple(uw0.shape)), const((1, lat)),
           const((lat, lat)), const((1, lat)), const(tuple(uw2.shape)),
           const((1, n_out))],
        out_specs=pl.BlockSpec((1, n_out), lambda i: (0, 0)),
        scratch_shapes=[pltpu.VMEM((8, lat), jnp.float32)],
    )(den, nodes_e, *parts_list, vwa, vwsr, cn, vw1, vb1_2, stats, ge,
      uw0, ub0_2, uw1, ub1_2, uw2, ub2_2)


def kernel(nodes, edges, senders, receivers, n_node, n_edge, na, nW, nb, ea,
           eW, eb, ga, gW, gb, eW0, eb0, eW1, eb1, vW0, vb0, vW1, vb1, uW0,
           ub0, uW1, ub1, uW2, ub2):
    f32 = jnp.float32
    n, _ = nodes.shape
    e, _ = edges.shape
    lat = nW.shape[1]

    g = jnp.concatenate([jnp.arcsinh(n_node.astype(f32)),
                         jnp.arcsinh(n_edge.astype(f32))]).reshape(1, 2)
    glob_e = (ga[0] * jnp.arcsinh(ga[1] * g + ga[2]) + ga[3]) @ gW + gb
    eW0a = eW0[0:lat]
    wsr_e = eW0[lat:2 * lat] + eW0[2 * lat:3 * lat]
    eW0g = eW0[3 * lat:4 * lat]
    wee = eW @ eW0a
    wee8 = jnp.kron(jnp.eye(8, dtype=f32), wee)
    edges_pk = edges.reshape(e // 8, 128)
    ce = (eb @ eW0a + glob_e[0] @ eW0g + eb0).reshape(1, lat)
    vwa = vW0[0:lat]
    vwsr = vW0[lat:2 * lat] + vW0[2 * lat:3 * lat]
    vW0g = vW0[3 * lat:4 * lat]
    cn = (glob_e[0] @ vW0g + vb0).reshape(1, lat)
    den = jnp.maximum(jnp.concatenate([n_node, n_edge]).astype(f32), 1.0)

    nodes_e, proj = _node_embed(na, nodes, nW, nb.reshape(1, lat), wsr_e)

    zeros = jnp.zeros((n, lat), f32)
    ec = e // K
    be = 6400
    nch_s = (-(-ec // (NW * CHS)) + NBUFS - 1) // NBUFS * NBUFS
    npad = NW * nch_s * CHS - ec
    parts_list, stats_list = [], []
    for k in range(K):
        sndk = senders[k * ec:(k + 1) * ec]
        gath_k = _sc_gather(proj, sndk.reshape(NW, ec // NW // CH, CH), ec, lat)
        eu_k, st_k = _edge_mlp(ea, edges_pk, gath_k, wee8, eW1, ce,
                               eb1.reshape(1, lat), be, k * (ec // be), ec)
        snd3k = jnp.concatenate([sndk, jnp.full((npad,), n, jnp.int32)]
                                ).reshape(NW, nch_s, CHS)
        parts_list.append(
            _sc_scatter(eu_k, snd3k, zeros, n, ec, lat, nch_s).reshape(NC, n, lat))
        stats_list.append(st_k)
    stats = jnp.concatenate(stats_list, axis=0)

    out = _node_update(den, nodes_e, parts_list, vwa, vwsr, cn,
                       vW1, vb1.reshape(1, lat), stats, glob_e,
                       uW0, ub0.reshape(1, -1), uW1, ub1.reshape(1, -1),
                       uW2, ub2.reshape(1, -1))
    return out.reshape(-1)
